# Initial kernel scaffold; baseline (speedup 1.0000x reference)
import functools

import jax
import jax.numpy as jnp
from jax import lax
from jax.experimental import pallas as pl
from jax.experimental.pallas import tpu as pltpu
from jax.experimental.pallas import tpu_sc as plsc

NC = 2
NS = 16
NW = NC * NS
CH = 80


def _make_spmm(n_nodes, n_edges, d):
    epw = n_edges // NW
    nch = epw // CH
    assert epw * NW == n_edges and nch * CH == epw
    rpt = n_nodes // NS
    zch = 125
    nz = rpt // zch
    assert rpt * NS == n_nodes and nz * zch == rpt
    mesh = plsc.VectorSubcoreMesh(core_axis_name="c", subcore_axis_name="s")

    @functools.partial(
        pl.kernel,
        out_type=jax.ShapeDtypeStruct((NC, n_nodes, d), jnp.float32),
        mesh=mesh,
        scratch_types=[
            pltpu.VMEM((epw,), jnp.int32),
            pltpu.VMEM((nch, CH), jnp.int32),
            pltpu.VMEM((epw,), jnp.float32),
            pltpu.VMEM((CH, d), jnp.float32),
            pltpu.VMEM((125, d), jnp.float32),
            pltpu.VMEM_SHARED((n_nodes, d), jnp.float32),
        ],
    )
    def spmm(h_hbm, src_hbm, dst_hbm, w_hbm, out_hbm,
             src_v, dst_v, w_v, rows_v, zero_v, acc):
        c = lax.axis_index("c")
        s = lax.axis_index("s")
        wid = s * NC + c
        pltpu.sync_copy(src_hbm.at[wid], src_v)
        pltpu.sync_copy(dst_hbm.at[wid], dst_v)
        pltpu.sync_copy(w_hbm.at[wid], w_v)

        zch = zero_v.shape[0]
        nz = rpt // zch
        zv = jnp.zeros((16,), jnp.float32)

        @pl.loop(0, zch)
        def _(r):
            for k in range(d // 16):
                zero_v[r, pl.ds(k * 16, 16)] = zv

        for q in range(nz):
            pltpu.sync_copy(zero_v, acc.at[pl.ds(s * rpt + q * zch, zch)])
        plsc.subcore_barrier()

        ids = lax.iota(jnp.int32, 16)

        @pl.loop(0, nch)
        def _(j):
            ebase = j * CH
            pltpu.sync_copy(h_hbm.at[src_v.at[pl.ds(ebase, CH)]], rows_v)
            for g in range(CH // 16):
                wv = w_v[pl.ds(ebase + g * 16, 16)]
                ev = ids + (g * 16)
                for f in range(d):
                    fv = jnp.full((16,), f, jnp.int32)
                    col = plsc.load_gather(rows_v, [ev, fv])
                    plsc.store_scatter(rows_v, [ev, fv], col * wv)
            pltpu.sync_copy(rows_v, acc.at[dst_v.at[j]], add=True)

        plsc.subcore_barrier()
        for q in range(nz):
            r0 = s * rpt + q * zch
            pltpu.sync_copy(acc.at[pl.ds(r0, zch)],
                            out_hbm.at[c, pl.ds(r0, zch)])

    return spmm


def _mid_tc(p, w0, w1):
    n = p.shape[1]

    def body(p_ref, w0_ref, w1_ref, o_ref):
        h = p_ref[0] + p_ref[1]
        h = jnp.maximum(
            jnp.dot(h, w0_ref[...], preferred_element_type=jnp.float32), 0.0)
        o_ref[...] = jnp.dot(h, w1_ref[...],
                             preferred_element_type=jnp.float32)

    return pl.pallas_call(
        body,
        out_shape=jax.ShapeDtypeStruct((n, w1.shape[1]), jnp.float32),
    )(p, w0, w1)


def _loss_tc(q, label, mask2d):
    def body(q_ref, y_ref, m_ref, loss_ref, acc_ref):
        logits = q_ref[0] + q_ref[1]
        y = y_ref[...]
        m = m_ref[...]
        k = logits.shape[1]
        rowmax = jnp.max(logits, axis=1, keepdims=True)
        sh = logits - rowmax
        lse = jnp.log(jnp.sum(jnp.exp(sh), axis=1, keepdims=True))
        ce = -jnp.sum(y * (sh - lse), axis=1, keepdims=True)
        ii = lax.broadcasted_iota(jnp.int32, logits.shape, 1)
        am_l = jnp.min(jnp.where(logits >= rowmax, ii, k), axis=1,
                       keepdims=True)
        ymax = jnp.max(y, axis=1, keepdims=True)
        am_y = jnp.min(jnp.where(y >= ymax, ii, k), axis=1, keepdims=True)
        correct = (am_l == am_y).astype(jnp.float32)
        msum = jnp.sum(m)
        loss_ref[0, 0] = jnp.sum(ce * m) / msum
        acc_ref[0, 0] = jnp.sum(correct * m) / msum

    return pl.pallas_call(
        body,
        out_shape=(jax.ShapeDtypeStruct((1, 1), jnp.float32),
                   jax.ShapeDtypeStruct((1, 1), jnp.float32)),
    )(q, label, mask2d)


def kernel(x, label, mask, edge_index, edge_weight, W0, W1):
    n, d_in = x.shape
    e = edge_index.shape[1]
    src = edge_index[0].astype(jnp.int32)
    dst = edge_index[1].astype(jnp.int32)
    epw = e // NW
    srcr = src.reshape(NW, epw)
    dstr = dst.reshape(NW, epw // CH, CH)
    wr = edge_weight.reshape(NW, epw)

    p = _make_spmm(n, e, d_in)(x, srcr, dstr, wr)
    h1 = _mid_tc(p, W0, W1)
    q = _make_spmm(n, e, h1.shape[1])(h1, srcr, dstr, wr)
    loss2, acc2 = _loss_tc(q, label, mask.reshape(n, 1))
    return (loss2[0, 0], acc2[0, 0])


# speedup vs baseline: 1.0102x; 1.0102x over previous
import dataclasses
import functools

import jax
import jax.numpy as jnp
from jax import lax
from jax.experimental import pallas as pl
from jax.experimental.pallas import tpu as pltpu
from jax.experimental.pallas import tpu_sc as plsc

NC = 2
NS = 16
NW = NC * NS
CH = 80


def _make_spmm(n_nodes, n_edges, d):
    epw = n_edges // NW
    ibc = 25
    ib = ibc * CH
    nib = epw // ib
    assert epw * NW == n_edges and nib * ib == epw
    nrch = n_nodes // CH
    assert nrch * CH == n_nodes
    mesh = plsc.VectorSubcoreMesh(core_axis_name="c", subcore_axis_name="s")
    cp = pltpu.CompilerParams()
    if "needs_layout_passes" in pltpu.CompilerParams.__dataclass_fields__:
        cp = dataclasses.replace(cp, needs_layout_passes=False)
    if d % 128 != 0:
        cp = dataclasses.replace(cp, use_tc_tiling_on_sc=False)

    @functools.partial(
        pl.kernel,
        out_type=jax.ShapeDtypeStruct((NC, n_nodes, d), jnp.float32),
        mesh=mesh,
        compiler_params=cp,
        scratch_types=[
            pltpu.VMEM((ib,), jnp.int32),
            pltpu.VMEM((ibc, CH), jnp.int32),
            pltpu.VMEM((ib,), jnp.float32),
            pltpu.VMEM((CH, d), jnp.float32),
            pltpu.VMEM((CH, d), jnp.float32),
            pltpu.VMEM_SHARED((n_nodes, d), jnp.float32),
        ],
    )
    def spmm(h_hbm, src_hbm, dst_hbm, w_hbm, out_hbm,
             src_v, dst_v, w_v, rows_v, scaled_v, acc):
        c = lax.axis_index("c")
        s = lax.axis_index("s")
        wid = s * NC + c

        zv = jnp.zeros((16,), jnp.float32)

        @pl.loop(0, CH)
        def _(r):
            for k in range(d // 16):
                rows_v[r, pl.ds(k * 16, 16)] = zv

        @pl.loop(s, nrch, step=NS)
        def _(q):
            pltpu.sync_copy(rows_v, acc.at[pl.ds(q * CH, CH)])
        plsc.subcore_barrier()

        ids = lax.iota(jnp.int32, 16)

        @pl.loop(0, nib)
        def _(b):
            blk = wid * nib + b
            pltpu.sync_copy(src_hbm.at[blk], src_v)
            pltpu.sync_copy(dst_hbm.at[blk], dst_v)
            pltpu.sync_copy(w_hbm.at[blk], w_v)

            @pl.loop(0, ibc)
            def _(j):
                ebase = j * CH
                pltpu.sync_copy(h_hbm.at[src_v.at[pl.ds(ebase, CH)]], rows_v)
                for g in range(CH // 16):
                    wv = w_v[pl.ds(ebase + g * 16, 16)]
                    ev = ids + (g * 16)
                    for f in range(d):
                        fv = jnp.full((16,), f, jnp.int32)
                        col = plsc.load_gather(rows_v, [ev, fv])
                        plsc.store_scatter(scaled_v, [ev, fv], col * wv)
                pltpu.sync_copy(scaled_v, acc.at[dst_v.at[j]], add=True)

        plsc.subcore_barrier()

        @pl.loop(s, nrch, step=NS)
        def _(q):
            pltpu.sync_copy(acc.at[pl.ds(q * CH, CH)],
                            out_hbm.at[c, pl.ds(q * CH, CH)])

    return spmm


def _mid_tc(p, w0):
    n = p.shape[1]

    def body(p_ref, w0_ref, o_ref):
        h = p_ref[0] + p_ref[1]
        o_ref[...] = jnp.maximum(
            jnp.dot(h, w0_ref[...], preferred_element_type=jnp.float32), 0.0)

    return pl.pallas_call(
        body,
        out_shape=jax.ShapeDtypeStruct((n, w0.shape[1]), jnp.float32),
    )(p, w0)


def _loss_tc(q, w1, label, mask2d):
    def body(q_ref, w1_ref, y_ref, m_ref, loss_ref, acc_ref):
        logits = jnp.dot(q_ref[0] + q_ref[1], w1_ref[...],
                         preferred_element_type=jnp.float32)
        y = y_ref[...]
        m = m_ref[...]
        k = logits.shape[1]
        rowmax = jnp.max(logits, axis=1, keepdims=True)
        sh = logits - rowmax
        lse = jnp.log(jnp.sum(jnp.exp(sh), axis=1, keepdims=True))
        ce = -jnp.sum(y * (sh - lse), axis=1, keepdims=True)
        ii = lax.broadcasted_iota(jnp.int32, logits.shape, 1)
        am_l = jnp.min(jnp.where(logits >= rowmax, ii, k), axis=1,
                       keepdims=True)
        ymax = jnp.max(y, axis=1, keepdims=True)
        am_y = jnp.min(jnp.where(y >= ymax, ii, k), axis=1, keepdims=True)
        correct = (am_l == am_y).astype(jnp.float32)
        msum = jnp.sum(m)
        loss_ref[...] = (jnp.sum(ce * m) / msum).reshape(1, 1)
        acc_ref[...] = (jnp.sum(correct * m) / msum).reshape(1, 1)

    return pl.pallas_call(
        body,
        out_shape=(jax.ShapeDtypeStruct((1, 1), jnp.float32),
                   jax.ShapeDtypeStruct((1, 1), jnp.float32)),
    )(q, w1, label, mask2d)


def kernel(x, label, mask, edge_index, edge_weight, W0, W1):
    n, d_in = x.shape
    e = edge_index.shape[1]
    src = edge_index[0].astype(jnp.int32)
    dst = edge_index[1].astype(jnp.int32)
    epw = e // NW
    ib = 25 * CH
    nblk = e // ib
    srcr = src.reshape(nblk, ib)
    dstr = dst.reshape(nblk, ib // CH, CH)
    wr = edge_weight.reshape(nblk, ib)

    spmm = _make_spmm(n, e, d_in)
    p = spmm(x, srcr, dstr, wr)
    h0 = _mid_tc(p, W0)
    q = spmm(h0, srcr, dstr, wr)
    loss2, acc2 = _loss_tc(q, W1, label, mask.reshape(n, 1))
    return (loss2[0, 0], acc2[0, 0])


# speedup vs baseline: 6.1928x; 6.1303x over previous
import dataclasses
import functools

import jax
import jax.numpy as jnp
from jax import lax
from jax.experimental import pallas as pl
from jax.experimental.pallas import tpu as pltpu
from jax.experimental.pallas import tpu_sc as plsc

NC = 2
NS = 16
NW = NC * NS
CH = 80


def _make_spmm(n_nodes, n_edges, d):
    epw = n_edges // NW
    ibc = 25
    ib = ibc * CH
    nib = epw // ib
    assert epw * NW == n_edges and nib * ib == epw
    nrch = n_nodes // CH
    assert nrch * CH == n_nodes
    mesh = plsc.VectorSubcoreMesh(core_axis_name="c", subcore_axis_name="s")
    cp = pltpu.CompilerParams()
    if "needs_layout_passes" in pltpu.CompilerParams.__dataclass_fields__:
        cp = dataclasses.replace(cp, needs_layout_passes=False)
    if d % 128 != 0:
        cp = dataclasses.replace(cp, use_tc_tiling_on_sc=False)

    @functools.partial(
        pl.kernel,
        out_type=jax.ShapeDtypeStruct((NC, n_nodes, d), jnp.float32),
        mesh=mesh,
        compiler_params=cp,
        scratch_types=[
            pltpu.VMEM((ib,), jnp.int32),
            pltpu.VMEM((ibc, CH), jnp.int32),
            pltpu.VMEM((ib,), jnp.float32),
            pltpu.VMEM((CH, d), jnp.float32),
            pltpu.VMEM((CH, d), jnp.float32),
            pltpu.VMEM_SHARED((n_nodes, d), jnp.float32),
        ],
    )
    def spmm(h_hbm, src_hbm, dst_hbm, w_hbm, out_hbm,
             src_v, dst_v, w_v, rows_v, scaled_v, acc):
        c = lax.axis_index("c")
        s = lax.axis_index("s")
        wid = s * NC + c

        zv = jnp.zeros((16,), jnp.float32)

        @pl.loop(0, CH)
        def _(r):
            for k in range(d // 16):
                rows_v[r, pl.ds(k * 16, 16)] = zv

        @pl.loop(s, nrch, step=NS)
        def _(q):
            pltpu.sync_copy(rows_v, acc.at[pl.ds(q * CH, CH)])
        plsc.subcore_barrier()

        dnums = lax.GatherDimensionNumbers(
            offset_dims=(), collapsed_slice_dims=(0,), start_index_map=(0,))

        def splat(vec, i):
            idx = jnp.full((16, 1), i, jnp.int32)
            return lax.gather(vec, idx, dnums, slice_sizes=(1,),
                              mode=lax.GatherScatterMode.PROMISE_IN_BOUNDS)

        @pl.loop(0, nib)
        def _(b):
            blk = wid * nib + b
            pltpu.sync_copy(src_hbm.at[blk], src_v)
            pltpu.sync_copy(dst_hbm.at[blk], dst_v)
            pltpu.sync_copy(w_hbm.at[blk], w_v)

            @pl.loop(0, ibc)
            def _(j):
                ebase = j * CH
                pltpu.sync_copy(h_hbm.at[src_v.at[pl.ds(ebase, CH)]], rows_v)
                for g in range(CH // 16):
                    wv = w_v[pl.ds(ebase + g * 16, 16)]
                    for i in range(16):
                        e = g * 16 + i
                        wb = splat(wv, i)
                        for k in range(d // 16):
                            sl = pl.ds(k * 16, 16)
                            scaled_v[e, sl] = rows_v[e, sl] * wb
                pltpu.sync_copy(scaled_v, acc.at[dst_v.at[j]], add=True)

        plsc.subcore_barrier()

        @pl.loop(s, nrch, step=NS)
        def _(q):
            pltpu.sync_copy(acc.at[pl.ds(q * CH, CH)],
                            out_hbm.at[c, pl.ds(q * CH, CH)])

    return spmm


def _mid_tc(p, w0):
    n = p.shape[1]

    def body(p_ref, w0_ref, o_ref):
        h = p_ref[0] + p_ref[1]
        o_ref[...] = jnp.maximum(
            jnp.dot(h, w0_ref[...], preferred_element_type=jnp.float32), 0.0)

    return pl.pallas_call(
        body,
        out_shape=jax.ShapeDtypeStruct((n, w0.shape[1]), jnp.float32),
    )(p, w0)


def _loss_tc(q, w1, label, mask2d):
    def body(q_ref, w1_ref, y_ref, m_ref, loss_ref, acc_ref):
        logits = jnp.dot(q_ref[0] + q_ref[1], w1_ref[...],
                         preferred_element_type=jnp.float32)
        y = y_ref[...]
        m = m_ref[...]
        k = logits.shape[1]
        rowmax = jnp.max(logits, axis=1, keepdims=True)
        sh = logits - rowmax
        lse = jnp.log(jnp.sum(jnp.exp(sh), axis=1, keepdims=True))
        ce = -jnp.sum(y * (sh - lse), axis=1, keepdims=True)
        ii = lax.broadcasted_iota(jnp.int32, logits.shape, 1)
        am_l = jnp.min(jnp.where(logits >= rowmax, ii, k), axis=1,
                       keepdims=True)
        ymax = jnp.max(y, axis=1, keepdims=True)
        am_y = jnp.min(jnp.where(y >= ymax, ii, k), axis=1, keepdims=True)
        correct = (am_l == am_y).astype(jnp.float32)
        msum = jnp.sum(m)
        loss_ref[...] = (jnp.sum(ce * m) / msum).reshape(1, 1)
        acc_ref[...] = (jnp.sum(correct * m) / msum).reshape(1, 1)

    return pl.pallas_call(
        body,
        out_shape=(jax.ShapeDtypeStruct((1, 1), jnp.float32),
                   jax.ShapeDtypeStruct((1, 1), jnp.float32)),
    )(q, w1, label, mask2d)


def kernel(x, label, mask, edge_index, edge_weight, W0, W1):
    n, d_in = x.shape
    e = edge_index.shape[1]
    src = edge_index[0].astype(jnp.int32)
    dst = edge_index[1].astype(jnp.int32)
    epw = e // NW
    ib = 25 * CH
    nblk = e // ib
    srcr = src.reshape(nblk, ib)
    dstr = dst.reshape(nblk, ib // CH, CH)
    wr = edge_weight.reshape(nblk, ib)

    spmm = _make_spmm(n, e, d_in)
    p = spmm(x, srcr, dstr, wr)
    h0 = _mid_tc(p, W0)
    q = spmm(h0, srcr, dstr, wr)
    loss2, acc2 = _loss_tc(q, W1, label, mask.reshape(n, 1))
    return (loss2[0, 0], acc2[0, 0])


# speedup vs baseline: 6.9212x; 1.1176x over previous
import dataclasses
import functools

import jax
import jax.numpy as jnp
from jax import lax
from jax.experimental import pallas as pl
from jax.experimental.pallas import tpu as pltpu
from jax.experimental.pallas import tpu_sc as plsc

NC = 2
NS = 16
NW = NC * NS
CH = 80


def _make_spmm(n_nodes, n_edges, d):
    epw = n_edges // NW
    ibc = 25
    ib = ibc * CH
    nib = epw // ib
    assert epw * NW == n_edges and nib * ib == epw
    nrch = n_nodes // CH
    assert nrch * CH == n_nodes
    mesh = plsc.VectorSubcoreMesh(core_axis_name="c", subcore_axis_name="s")
    cp = pltpu.CompilerParams()
    if "needs_layout_passes" in pltpu.CompilerParams.__dataclass_fields__:
        cp = dataclasses.replace(cp, needs_layout_passes=False)
    if d % 128 != 0:
        cp = dataclasses.replace(cp, use_tc_tiling_on_sc=False)

    @functools.partial(
        pl.kernel,
        out_type=jax.ShapeDtypeStruct((NC, n_nodes, d), jnp.float32),
        mesh=mesh,
        compiler_params=cp,
        scratch_types=[
            pltpu.VMEM((ib,), jnp.int32),
            pltpu.VMEM((ibc, CH), jnp.int32),
            pltpu.VMEM((ib,), jnp.float32),
            pltpu.VMEM((CH, d), jnp.float32),
            pltpu.VMEM((CH, d), jnp.float32),
            pltpu.VMEM_SHARED((n_nodes, d), jnp.float32),
        ],
    )
    def spmm(h_hbm, src_hbm, dst_hbm, w_hbm, out_hbm,
             src_v, dst_v, w_v, rows_v, scaled_v, acc):
        c = lax.axis_index("c")
        s = lax.axis_index("s")
        wid = s * NC + c

        zv = jnp.zeros((16,), jnp.float32)

        @pl.loop(0, CH)
        def _(r):
            for k in range(d // 16):
                rows_v[r, pl.ds(k * 16, 16)] = zv

        @pl.loop(s, nrch, step=NS)
        def _(q):
            pltpu.sync_copy(rows_v, acc.at[pl.ds(q * CH, CH)])
        plsc.subcore_barrier()

        dnums = lax.GatherDimensionNumbers(
            offset_dims=(), collapsed_slice_dims=(0,), start_index_map=(0,))

        def splat(vec, i):
            idx = jnp.full((16, 1), i, jnp.int32)
            return lax.gather(vec, idx, dnums, slice_sizes=(1,),
                              mode=lax.GatherScatterMode.PROMISE_IN_BOUNDS)

        @pl.loop(0, nib)
        def _(b):
            blk = wid * nib + b
            pltpu.sync_copy(src_hbm.at[blk], src_v)
            pltpu.sync_copy(dst_hbm.at[blk], dst_v)
            pltpu.sync_copy(w_hbm.at[blk], w_v)

            @pl.loop(0, ibc)
            def _(j):
                ebase = j * CH
                pltpu.sync_copy(h_hbm.at[src_v.at[pl.ds(ebase, CH)]], rows_v)
                for g in range(CH // 16):
                    wv = w_v[pl.ds(ebase + g * 16, 16)]
                    for i in range(16):
                        e = g * 16 + i
                        wb = splat(wv, i)
                        for k in range(d // 16):
                            sl = pl.ds(k * 16, 16)
                            scaled_v[e, sl] = rows_v[e, sl] * wb
                pltpu.sync_copy(scaled_v, acc.at[dst_v.at[j]], add=True)

        plsc.subcore_barrier()

        @pl.loop(s, nrch, step=NS)
        def _(q):
            pltpu.sync_copy(acc.at[pl.ds(q * CH, CH)],
                            out_hbm.at[c, pl.ds(q * CH, CH)])

    return spmm


def _matmul_tc(x, w):
    def body(x_ref, w_ref, o_ref):
        o_ref[...] = jnp.dot(x_ref[...], w_ref[...],
                             preferred_element_type=jnp.float32)

    return pl.pallas_call(
        body,
        out_shape=jax.ShapeDtypeStruct((x.shape[0], w.shape[1]), jnp.float32),
    )(x, w)


def _mid_tc(p, w1):
    n = p.shape[1]

    def body(p_ref, w1_ref, o_ref):
        h = jnp.maximum(p_ref[0] + p_ref[1], 0.0)
        o_ref[...] = jnp.dot(h, w1_ref[...],
                             preferred_element_type=jnp.float32)

    return pl.pallas_call(
        body,
        out_shape=jax.ShapeDtypeStruct((n, w1.shape[1]), jnp.float32),
    )(p, w1)


def _loss_tc(q, label, mask2d):
    def body(q_ref, y_ref, m_ref, loss_ref, acc_ref):
        logits = q_ref[0] + q_ref[1]
        y = y_ref[...]
        m = m_ref[...]
        k = logits.shape[1]
        rowmax = jnp.max(logits, axis=1, keepdims=True)
        sh = logits - rowmax
        lse = jnp.log(jnp.sum(jnp.exp(sh), axis=1, keepdims=True))
        ce = -jnp.sum(y * (sh - lse), axis=1, keepdims=True)
        ii = lax.broadcasted_iota(jnp.int32, logits.shape, 1)
        am_l = jnp.min(jnp.where(logits >= rowmax, ii, k), axis=1,
                       keepdims=True)
        ymax = jnp.max(y, axis=1, keepdims=True)
        am_y = jnp.min(jnp.where(y >= ymax, ii, k), axis=1, keepdims=True)
        correct = (am_l == am_y).astype(jnp.float32)
        msum = jnp.sum(m)
        loss_ref[...] = (jnp.sum(ce * m) / msum).reshape(1, 1)
        acc_ref[...] = (jnp.sum(correct * m) / msum).reshape(1, 1)

    return pl.pallas_call(
        body,
        out_shape=(jax.ShapeDtypeStruct((1, 1), jnp.float32),
                   jax.ShapeDtypeStruct((1, 1), jnp.float32)),
    )(q, label, mask2d)


def kernel(x, label, mask, edge_index, edge_weight, W0, W1):
    n, d_in = x.shape
    e = edge_index.shape[1]
    src = edge_index[0].astype(jnp.int32)
    dst = edge_index[1].astype(jnp.int32)
    epw = e // NW
    ib = 25 * CH
    nblk = e // ib
    srcr = src.reshape(nblk, ib)
    dstr = dst.reshape(nblk, ib // CH, CH)
    wr = edge_weight.reshape(nblk, ib)

    xw = _matmul_tc(x, W0)
    p = _make_spmm(n, e, W0.shape[1])(xw, srcr, dstr, wr)
    h1 = _mid_tc(p, W1)
    q = _make_spmm(n, e, W1.shape[1])(h1, srcr, dstr, wr)
    loss2, acc2 = _loss_tc(q, label, mask.reshape(n, 1))
    return (loss2[0, 0], acc2[0, 0])


# speedup vs baseline: 10.1614x; 1.4682x over previous
import dataclasses
import functools

import jax
import jax.numpy as jnp
from jax import lax
from jax.experimental import pallas as pl
from jax.experimental.pallas import tpu as pltpu
from jax.experimental.pallas import tpu_sc as plsc

NC = 2
NS = 16
NW = NC * NS
CH = 80


def _make_spmm(n_nodes, n_edges, d):
    epw = n_edges // NW
    ibc = 25
    ib = ibc * CH
    nib = epw // ib
    assert epw * NW == n_edges and nib * ib == epw
    nrch = n_nodes // CH
    assert nrch * CH == n_nodes
    mesh = plsc.VectorSubcoreMesh(core_axis_name="c", subcore_axis_name="s")
    cp = pltpu.CompilerParams()
    if "needs_layout_passes" in pltpu.CompilerParams.__dataclass_fields__:
        cp = dataclasses.replace(cp, needs_layout_passes=False)
    if d % 128 != 0:
        cp = dataclasses.replace(cp, use_tc_tiling_on_sc=False)

    @functools.partial(
        pl.kernel,
        out_type=jax.ShapeDtypeStruct((NC, n_nodes, d), jnp.float32),
        mesh=mesh,
        compiler_params=cp,
        scratch_types=[
            pltpu.VMEM((2 * ibc, CH), jnp.int32),
            pltpu.VMEM((2 * ibc, CH), jnp.int32),
            pltpu.VMEM((2 * ibc, CH), jnp.float32),
            pltpu.VMEM((2, CH, d), jnp.float32),
            pltpu.VMEM_SHARED((n_nodes, d), jnp.float32),
            pltpu.SemaphoreType.DMA,
            pltpu.SemaphoreType.DMA,
            pltpu.SemaphoreType.DMA,
        ],
    )
    def spmm(h_hbm, src_hbm, dst_hbm, w_hbm, out_hbm,
             src_v, dst_v, w_v, rows_v, acc, isem, gsem0, gsem1):
        c = lax.axis_index("c")
        s = lax.axis_index("s")
        wid = s * NC + c
        gsems = (gsem0, gsem1)

        def stage(sl, b):
            blk = wid * nib + b
            pltpu.async_copy(src_hbm.at[blk], src_v.at[pl.ds(sl * ibc, ibc)],
                             isem)
            pltpu.async_copy(dst_hbm.at[blk], dst_v.at[pl.ds(sl * ibc, ibc)],
                             isem)
            pltpu.async_copy(w_hbm.at[blk], w_v.at[pl.ds(sl * ibc, ibc)],
                             isem)

        def wait_stage(sl):
            pltpu.make_async_copy(src_hbm.at[0],
                                  src_v.at[pl.ds(sl * ibc, ibc)], isem).wait()
            pltpu.make_async_copy(dst_hbm.at[0],
                                  dst_v.at[pl.ds(sl * ibc, ibc)], isem).wait()
            pltpu.make_async_copy(w_hbm.at[0],
                                  w_v.at[pl.ds(sl * ibc, ibc)], isem).wait()

        def issue_gather(sl, j, p):
            pltpu.async_copy(h_hbm.at[src_v.at[sl * ibc + j]],
                             rows_v.at[p], gsems[p])

        def wait_gather(p):
            pltpu.make_async_copy(out_hbm.at[0, pl.ds(0, CH)],
                                  rows_v.at[p], gsems[p]).wait()

        stage(0, 0)

        zv = jnp.zeros((16,), jnp.float32)

        @pl.loop(0, CH)
        def _(r):
            for k in range(d // 16):
                rows_v[0, r, pl.ds(k * 16, 16)] = zv

        @pl.loop(s, nrch, step=NS)
        def _(q):
            pltpu.sync_copy(rows_v.at[0], acc.at[pl.ds(q * CH, CH)])
        plsc.subcore_barrier()

        dnums = lax.GatherDimensionNumbers(
            offset_dims=(), collapsed_slice_dims=(0,), start_index_map=(0,))

        def splat(vec, i):
            idx = jnp.full((16, 1), i, jnp.int32)
            return lax.gather(vec, idx, dnums, slice_sizes=(1,),
                              mode=lax.GatherScatterMode.PROMISE_IN_BOUNDS)

        def scale_scatter(sl, j, p):
            wait_gather(p)
            for g in range(CH // 16):
                wv = w_v[sl * ibc + j, pl.ds(g * 16, 16)]
                for i in range(16):
                    e = g * 16 + i
                    wb = splat(wv, i)
                    for k in range(d // 16):
                        slc = pl.ds(k * 16, 16)
                        rows_v[p, e, slc] = rows_v[p, e, slc] * wb
            pltpu.sync_copy(rows_v.at[p], acc.at[dst_v.at[sl * ibc + j]],
                            add=True)

        @pl.loop(0, nib)
        def _(b):
            sl = b % 2
            wait_stage(sl)

            @pl.when(b + 1 < nib)
            def _():
                stage(1 - sl, b + 1)

            issue_gather(sl, 0, 0)
            issue_gather(sl, 1, 1)

            @pl.loop(0, ibc - 1, step=2)
            def _(j):
                scale_scatter(sl, j, 0)
                issue_gather(sl, j + 2, 0)
                scale_scatter(sl, j + 1, 1)

                @pl.when(j + 3 < ibc)
                def _():
                    issue_gather(sl, j + 3, 1)

            scale_scatter(sl, ibc - 1, 0)

        plsc.subcore_barrier()

        @pl.loop(s, nrch, step=NS)
        def _(q):
            pltpu.sync_copy(acc.at[pl.ds(q * CH, CH)],
                            out_hbm.at[c, pl.ds(q * CH, CH)])

    return spmm


def _matmul_tc(x, w):
    def body(x_ref, w_ref, o_ref):
        o_ref[...] = jnp.dot(x_ref[...], w_ref[...],
                             preferred_element_type=jnp.float32)

    return pl.pallas_call(
        body,
        out_shape=jax.ShapeDtypeStruct((x.shape[0], w.shape[1]), jnp.float32),
    )(x, w)


def _mid_tc(p, w1):
    n = p.shape[1]

    def body(p_ref, w1_ref, o_ref):
        h = jnp.maximum(p_ref[0] + p_ref[1], 0.0)
        o_ref[...] = jnp.dot(h, w1_ref[...],
                             preferred_element_type=jnp.float32)

    return pl.pallas_call(
        body,
        out_shape=jax.ShapeDtypeStruct((n, w1.shape[1]), jnp.float32),
    )(p, w1)


def _loss_tc(q, label, mask2d):
    def body(q_ref, y_ref, m_ref, loss_ref, acc_ref):
        logits = q_ref[0] + q_ref[1]
        y = y_ref[...]
        m = m_ref[...]
        k = logits.shape[1]
        rowmax = jnp.max(logits, axis=1, keepdims=True)
        sh = logits - rowmax
        lse = jnp.log(jnp.sum(jnp.exp(sh), axis=1, keepdims=True))
        ce = -jnp.sum(y * (sh - lse), axis=1, keepdims=True)
        ii = lax.broadcasted_iota(jnp.int32, logits.shape, 1)
        am_l = jnp.min(jnp.where(logits >= rowmax, ii, k), axis=1,
                       keepdims=True)
        ymax = jnp.max(y, axis=1, keepdims=True)
        am_y = jnp.min(jnp.where(y >= ymax, ii, k), axis=1, keepdims=True)
        correct = (am_l == am_y).astype(jnp.float32)
        msum = jnp.sum(m)
        loss_ref[...] = (jnp.sum(ce * m) / msum).reshape(1, 1)
        acc_ref[...] = (jnp.sum(correct * m) / msum).reshape(1, 1)

    return pl.pallas_call(
        body,
        out_shape=(jax.ShapeDtypeStruct((1, 1), jnp.float32),
                   jax.ShapeDtypeStruct((1, 1), jnp.float32)),
    )(q, label, mask2d)


def kernel(x, label, mask, edge_index, edge_weight, W0, W1):
    n, d_in = x.shape
    e = edge_index.shape[1]
    src = edge_index[0].astype(jnp.int32)
    dst = edge_index[1].astype(jnp.int32)
    epw = e // NW
    ibc = 25
    nblk = e // (ibc * CH)
    srcr = src.reshape(nblk, ibc, CH)
    dstr = dst.reshape(nblk, ibc, CH)
    wr = edge_weight.reshape(nblk, ibc, CH)

    xw = _matmul_tc(x, W0)
    p = _make_spmm(n, e, W0.shape[1])(xw, srcr, dstr, wr)
    h1 = _mid_tc(p, W1)
    q = _make_spmm(n, e, W1.shape[1])(h1, srcr, dstr, wr)
    loss2, acc2 = _loss_tc(q, label, mask.reshape(n, 1))
    return (loss2[0, 0], acc2[0, 0])
